# baseline (speedup 1.0000x reference)
import functools

import jax
import jax.numpy as jnp
from jax import lax
from jax.experimental import pallas as pl
from jax.experimental.pallas import tpu as pltpu
from jax.experimental.pallas import tpu_sc as plsc

_REG = 0.001
_B = 16384
_D = 64
_NC = 2
_NS = 16
_L = 16
_NW = _NC * _NS
_BPW = _B // _NW
_CHUNK = 128
_NCHUNK = _BPW // _CHUNK


def _sc_body(user_hbm, ii_hbm, ij_hbm, eu_hbm, ei_hbm,
             pi_hbm, pj_hbm, reg_hbm,
             uidx_v, iidx_v, jidx_v, u_v, vi_v, vj_v, pi_v, pj_v, reg_v,
             sem):
    c = lax.axis_index("c")
    s = lax.axis_index("s")
    wid = s * _NC + c
    base = wid * _BPW

    pltpu.sync_copy(user_hbm.at[pl.ds(base, _BPW)], uidx_v)
    pltpu.sync_copy(ii_hbm.at[pl.ds(base, _BPW)], iidx_v)
    pltpu.sync_copy(ij_hbm.at[pl.ds(base, _BPW)], jidx_v)

    copies = []
    for k in range(_NCHUNK):
        sl = pl.ds(k * _CHUNK, _CHUNK)
        copies.append(pltpu.async_copy(
            eu_hbm.at[uidx_v.at[sl]], u_v.at[sl, :], sem))
        copies.append(pltpu.async_copy(
            ei_hbm.at[iidx_v.at[sl]], vi_v.at[sl, :], sem))
        copies.append(pltpu.async_copy(
            ei_hbm.at[jidx_v.at[sl]], vj_v.at[sl, :], sem))
    for cp in copies:
        cp.wait()

    lane = lax.iota(jnp.int32, _L)

    def group(g, reg_acc):
        acc_i = jnp.zeros((_L,), jnp.float32)
        acc_j = jnp.zeros((_L,), jnp.float32)
        for l in range(_L):
            r = g * _L + l
            ss_i = jnp.zeros((_L,), jnp.float32)
            ss_j = jnp.zeros((_L,), jnp.float32)
            for cc in range(_D // _L):
                col = pl.ds(cc * _L, _L)
                uu = u_v[r, col]
                vv = vi_v[r, col]
                ww = vj_v[r, col]
                ss_i = ss_i + uu * vv
                ss_j = ss_j + uu * ww
                reg_acc = reg_acc + uu * uu + vv * vv + ww * ww
            acc_i = jnp.where(lane == l, jnp.sum(ss_i), acc_i)
            acc_j = jnp.where(lane == l, jnp.sum(ss_j), acc_j)
        row = pl.ds(g * _L, _L)
        pi_v[row] = acc_i
        pj_v[row] = acc_j
        return reg_acc

    reg_acc = lax.fori_loop(0, _BPW // _L, group,
                            jnp.zeros((_L,), jnp.float32))
    reg_v[...] = reg_acc

    pltpu.sync_copy(pi_v, pi_hbm.at[pl.ds(base, _BPW)])
    pltpu.sync_copy(pj_v, pj_hbm.at[pl.ds(base, _BPW)])
    pltpu.sync_copy(reg_v, reg_hbm.at[wid])


_sc_call = functools.partial(
    pl.kernel,
    out_type=(
        jax.ShapeDtypeStruct((_B,), jnp.float32),
        jax.ShapeDtypeStruct((_B,), jnp.float32),
        jax.ShapeDtypeStruct((_NW, _L), jnp.float32),
    ),
    mesh=plsc.VectorSubcoreMesh(
        core_axis_name="c", subcore_axis_name="s",
        num_cores=_NC, num_subcores=_NS),
    compiler_params=pltpu.CompilerParams(
        needs_layout_passes=False, use_tc_tiling_on_sc=False),
    scratch_types=[
        pltpu.VMEM((_BPW,), jnp.int32),
        pltpu.VMEM((_BPW,), jnp.int32),
        pltpu.VMEM((_BPW,), jnp.int32),
        pltpu.VMEM((_BPW, _D), jnp.float32),
        pltpu.VMEM((_BPW, _D), jnp.float32),
        pltpu.VMEM((_BPW, _D), jnp.float32),
        pltpu.VMEM((_BPW,), jnp.float32),
        pltpu.VMEM((_BPW,), jnp.float32),
        pltpu.VMEM((_L,), jnp.float32),
        pltpu.SemaphoreType.DMA,
    ],
)(_sc_body)


def _loss_body(pi_ref, pj_ref, reg_ref, out_ref):
    x = pi_ref[...] - pj_ref[...]
    ls = jnp.minimum(x, 0.0) - jnp.log(1.0 + jnp.exp(-jnp.abs(x)))
    out_ref[0, 0] = _REG * jnp.sum(reg_ref[...]) - jnp.sum(ls)


_loss_call = pl.pallas_call(
    _loss_body,
    out_shape=jax.ShapeDtypeStruct((1, 1), jnp.float32),
    out_specs=pl.BlockSpec(memory_space=pltpu.SMEM),
)


def kernel(user, item_i, item_j, embed_user, embed_item):
    pi, pj, regp = _sc_call(user, item_i, item_j, embed_user, embed_item)
    loss = _loss_call(pi.reshape(_B // 128, 128),
                      pj.reshape(_B // 128, 128), regp)[0, 0]
    return (pi, pj, loss)
